# baseline (speedup 1.0000x reference)
import functools

import jax
import jax.numpy as jnp
import numpy as np
from jax import lax
from jax.experimental import pallas as pl
from jax.experimental.pallas import tpu as pltpu
from jax.experimental.pallas import tpu_sc as plsc

_SCALE = np.float32(np.sqrt(3) / 2.0)
_INV_SCALE = np.float32(1.0) / _SCALE
_MAGIC = np.float32(2.0**23)

_N_WORDS = 8388608
_NW = 32
_PER_W = _N_WORDS // _NW
_CHUNK = 16384
_N_CHUNKS = _PER_W // _CHUNK


def _round_rtne(x):
    c = jnp.where(x < 0.0, -_MAGIC, _MAGIC)
    r = (x + c) - c
    return jnp.where(jnp.abs(x) >= _MAGIC, x, r)


def _quant_pair(buf, off):
    va = buf[pl.ds(off, 16)]
    vb = buf[pl.ds(off + 128, 16)]
    ri = _round_rtne(vb * _INV_SCALE)
    y1 = ri * _SCALE
    odd = lax.rem(ri, jnp.float32(2.0)) != 0.0
    t = jnp.where(odd, jnp.float32(0.5), jnp.float32(0.0))
    y0 = _round_rtne(va + t) - t
    buf[pl.ds(off, 16)] = y0
    buf[pl.ds(off + 128, 16)] = y1


def _sc_body(x_hbm, o_hbm, buf0, buf1, sin0, sin1, sout0, sout1):
    wid = lax.axis_index("s") * 2 + lax.axis_index("c")
    base = wid * _PER_W
    bufs = (buf0, buf1)
    sins = (sin0, sin1)
    souts = (sout0, sout1)

    pltpu.async_copy(x_hbm.at[pl.ds(base, _CHUNK)], buf0, sin0)

    def chunk_step(g, carry):
        del carry
        slot = lax.rem(g, 2)

        def run_slot(s):
            buf, sin, sout = bufs[s], sins[s], souts[s]
            pltpu.make_async_copy(x_hbm.at[pl.ds(base, _CHUNK)], buf, sin).wait()

            @pl.when(g + 1 < _N_CHUNKS)
            def _():
                nbuf, nsin, nsout = bufs[1 - s], sins[1 - s], souts[1 - s]

                @pl.when(g >= 1)
                def _():
                    pltpu.make_async_copy(
                        nbuf, o_hbm.at[pl.ds(base, _CHUNK)], nsout
                    ).wait()

                pltpu.async_copy(
                    x_hbm.at[pl.ds(base + (g + 1) * _CHUNK, _CHUNK)], nbuf, nsin
                )

            def blk(b, c2):
                del c2
                off = b * 256
                for j in range(8):
                    _quant_pair(buf, off + j * 16)
                return 0

            lax.fori_loop(0, _CHUNK // 256, blk, 0)
            pltpu.async_copy(buf, o_hbm.at[pl.ds(base + g * _CHUNK, _CHUNK)], sout)

        @pl.when(slot == 0)
        def _():
            run_slot(0)

        @pl.when(slot == 1)
        def _():
            run_slot(1)

        return 0

    lax.fori_loop(0, _N_CHUNKS, chunk_step, 0)
    pltpu.make_async_copy(
        buf0, o_hbm.at[pl.ds(base, _CHUNK)], sout0
    ).wait()
    pltpu.make_async_copy(
        buf1, o_hbm.at[pl.ds(base, _CHUNK)], sout1
    ).wait()


def kernel(x, G):
    del G
    n = x.shape[0]
    a = x.reshape(n // 128, 128, 2).transpose(0, 2, 1).reshape(_N_WORDS)
    mesh = plsc.VectorSubcoreMesh(core_axis_name="c", subcore_axis_name="s")
    f = functools.partial(
        pl.kernel,
        mesh=mesh,
        out_type=jax.ShapeDtypeStruct((_N_WORDS,), jnp.float32),
        scratch_types=[
            pltpu.VMEM((_CHUNK,), jnp.float32),
            pltpu.VMEM((_CHUNK,), jnp.float32),
            pltpu.SemaphoreType.DMA,
            pltpu.SemaphoreType.DMA,
            pltpu.SemaphoreType.DMA,
            pltpu.SemaphoreType.DMA,
        ],
    )(_sc_body)
    y = f(a)
    return y.reshape(n // 128, 2, 128).transpose(0, 2, 1).reshape(n, 2)


# speedup vs baseline: 3.3492x; 3.3492x over previous
import jax
import jax.numpy as jnp
import numpy as np
from jax.experimental import pallas as pl
from jax.experimental.pallas import tpu as pltpu

_SCALE = np.float32(np.sqrt(3) / 2.0)
_INV_SCALE = np.float32(1.0) / _SCALE
_ONE_MINUS_SCALE = np.float32(1.0) - _SCALE

_ROWS = 65536
_COLS = 128
_BLOCK_ROWS = 16384
_SUB = 8


def _quant_body(x_ref, o_ref):
    v = x_ref[...]
    sub = jax.lax.broadcasted_iota(jnp.int32, v.shape, 1)
    is_x0 = (sub & 1) == 0
    ri = jnp.round(v * _INV_SCALE)
    ri_n = jnp.roll(ri, -1, axis=1)
    h = ri_n * jnp.float32(0.5)
    t = h - jnp.floor(h)
    a_sel = jnp.where(is_x0, jnp.round(v + t), ri)
    b_sel = jnp.where(is_x0, t, ri * _ONE_MINUS_SCALE)
    o_ref[...] = a_sel - b_sel


def kernel(x, G):
    del G
    n = x.shape[0]
    a = x.reshape(n // _COLS, _COLS, 2).transpose(0, 2, 1)
    a = a.reshape(_ROWS // _SUB, _SUB, _COLS)
    y = pl.pallas_call(
        _quant_body,
        grid=(_ROWS // _BLOCK_ROWS,),
        in_specs=[pl.BlockSpec((_BLOCK_ROWS // _SUB, _SUB, _COLS), lambda i: (i, 0, 0))],
        out_specs=pl.BlockSpec((_BLOCK_ROWS // _SUB, _SUB, _COLS), lambda i: (i, 0, 0)),
        out_shape=jax.ShapeDtypeStruct((_ROWS // _SUB, _SUB, _COLS), jnp.float32),
        compiler_params=pltpu.CompilerParams(
            dimension_semantics=("parallel",),
        ),
    )(a)
    return y.reshape(n // _COLS, 2, _COLS).transpose(0, 2, 1).reshape(n, 2)


# speedup vs baseline: 3.4251x; 1.0227x over previous
import jax
import jax.numpy as jnp
import numpy as np
from jax.experimental import pallas as pl
from jax.experimental.pallas import tpu as pltpu

_SCALE = np.float32(np.sqrt(3) / 2.0)
_INV_SCALE = np.float32(1.0) / _SCALE

_ROWS = 65536
_COLS = 128
_BLOCK_ROWS = 16384
_SUB = 8


def _quant_body(x_ref, o_ref):
    v = x_ref[...]
    sub = jax.lax.broadcasted_iota(jnp.int32, v.shape, 1)
    is_x0 = (sub & 1) == 0
    ri = jnp.round(v * _INV_SCALE)
    ri_n = jnp.roll(ri, -1, axis=1)
    h = ri_n * jnp.float32(0.5)
    t = h - jnp.floor(h)
    y0 = jnp.round(v + t) - t
    y1 = ri * _SCALE
    o_ref[...] = jnp.where(is_x0, y0, y1)


def kernel(x, G):
    del G
    n = x.shape[0]
    a = x.reshape(n // _COLS, _COLS, 2).transpose(0, 2, 1)
    a = a.reshape(_ROWS // _SUB, _SUB, _COLS)
    y = pl.pallas_call(
        _quant_body,
        grid=(_ROWS // _BLOCK_ROWS,),
        in_specs=[pl.BlockSpec((_BLOCK_ROWS // _SUB, _SUB, _COLS), lambda i: (i, 0, 0))],
        out_specs=pl.BlockSpec((_BLOCK_ROWS // _SUB, _SUB, _COLS), lambda i: (i, 0, 0)),
        out_shape=jax.ShapeDtypeStruct((_ROWS // _SUB, _SUB, _COLS), jnp.float32),
        compiler_params=pltpu.CompilerParams(
            dimension_semantics=("parallel",),
        ),
    )(a)
    return y.reshape(n // _COLS, 2, _COLS).transpose(0, 2, 1).reshape(n, 2)
